# baseline (speedup 1.0000x reference)
import functools

import jax
import jax.numpy as jnp
from jax import lax
from jax.experimental import pallas as pl
from jax.experimental.pallas import tpu as pltpu
from jax.experimental.pallas import tpu_sc as plsc

DIM = 64
NC = 2
NS = 16
NW = NC * NS
GROUP = 128
LANES = 16


@functools.lru_cache(maxsize=None)
def _make_kernel(W: int, B: int):
    mesh = plsc.VectorSubcoreMesh(core_axis_name="c", subcore_axis_name="s")

    @functools.partial(
        pl.kernel,
        mesh=mesh,
        out_type=jax.ShapeDtypeStruct((W, B, DIM), jnp.float32),
        scratch_types=[
            pltpu.VMEM((W, GROUP), jnp.int32),
            pltpu.VMEM((2, GROUP, DIM), jnp.float32),
            pltpu.VMEM((2 * GROUP,), jnp.int32),
            pltpu.SemaphoreType.DMA((2,)),
            pltpu.SemaphoreType.DMA((2,)),
        ],
        compiler_params=pltpu.CompilerParams(
            use_tc_tiling_on_sc=False, needs_layout_passes=False
        ),
    )
    def emb(idx_hbm, table_hbm, out_hbm, idx_v, rows_v, zpos_v, gsem, ssem):
        wid = lax.axis_index("s") * NC + lax.axis_index("c")
        col = wid * GROUP
        pltpu.sync_copy(idx_hbm.at[:, pl.ds(col, GROUP)], idx_v)

        def start_gather(r):
            b = r % 2
            pltpu.async_copy(table_hbm.at[idx_v.at[r]], rows_v.at[b], gsem.at[b])

        def wait_gather(r):
            b = r % 2
            pltpu.make_async_copy(
                table_hbm.at[idx_v.at[r]], rows_v.at[b], gsem.at[b]
            ).wait()

        def fix_zeros(r):
            b = r % 2
            cnt = jnp.int32(0)
            for s in range(GROUP // LANES):
                idxs = idx_v[r, pl.ds(s * LANES, LANES)]
                m0 = idxs == 0
                pos16 = lax.iota(jnp.int32, LANES) + (s * LANES)
                plsc.store_compressed(zpos_v.at[pl.ds(cnt, LANES)], pos16, mask=m0)
                cnt = cnt + jnp.sum(m0.astype(jnp.int32))

            zeros = jnp.zeros((LANES,), jnp.float32)

            def zero_one(i, carry):
                p = jnp.max(
                    plsc.load_gather(zpos_v, [jnp.full((LANES,), i, jnp.int32)])
                )
                for a in range(DIM // LANES):
                    rows_v[b, p, pl.ds(a * LANES, LANES)] = zeros
                return carry

            lax.fori_loop(0, cnt, zero_one, jnp.int32(0))

        def start_store(r):
            b = r % 2
            pltpu.async_copy(
                rows_v.at[b], out_hbm.at[r, pl.ds(col, GROUP)], ssem.at[b]
            )

        def wait_store(r):
            b = r % 2
            pltpu.make_async_copy(
                rows_v.at[b], out_hbm.at[r, pl.ds(col, GROUP)], ssem.at[b]
            ).wait()

        start_gather(0)
        for r in range(W):
            wait_gather(r)
            fix_zeros(r)
            if r + 1 < W:
                if r >= 1:
                    wait_store(r - 1)
                start_gather(r + 1)
            start_store(r)
        if W >= 2:
            wait_store(W - 2)
        wait_store(W - 1)

    return emb


def kernel(input, table):
    batch, width = input.shape
    out = _make_kernel(width, batch)(input.T, table)
    return out.transpose(1, 0, 2)
